# baseline (speedup 1.0000x reference)
import jax
import jax.numpy as jnp
from jax import lax
from jax.experimental import pallas as pl
from jax.experimental.pallas import tpu as pltpu
from jax.experimental.pallas import tpu_sc as plsc

VOCAB = 1000
NC, NS = 2, 16
NW = NC * NS
CHUNK = 32
BUFW = CHUNK * VOCAB


def _sc_body(x_hbm, z_hbm, out_hbm, idx_v, buf0, buf1, sem0, sem1):
    n_rows = x_hbm.shape[0]
    rpw = n_rows // NW
    nch = rpw // CHUNK
    wid = lax.axis_index("s") * NC + lax.axis_index("c")
    base_row = wid * rpw
    base_off = base_row * VOCAB

    pltpu.sync_copy(x_hbm.at[pl.ds(base_row, rpw)], idx_v.at[pl.ds(0, rpw)])
    pltpu.async_copy(z_hbm, buf0, sem0).wait()
    pltpu.async_copy(z_hbm, buf1, sem1).wait()

    lane = lax.iota(jnp.int32, 16)
    ones = jnp.full((16,), 1.0, jnp.float32)
    zf = jnp.zeros((16,), jnp.float32)

    def scatter(buf, c, val):
        for k in range(CHUNK // 16):
            col = idx_v[pl.ds(c * CHUNK + k * 16, 16)]
            pos = (lane + (k * 16)) * VOCAB + col
            plsc.store_scatter(buf, [pos], val)

    def dma(buf, sem, c):
        return pltpu.make_async_copy(
            buf, out_hbm.at[pl.ds(base_off + c * BUFW, BUFW)], sem)

    scatter(buf0, 0, ones)
    dma(buf0, sem0, 0).start()
    scatter(buf1, 1, ones)
    dma(buf1, sem1, 1).start()

    def pair(p, carry):
        for b, (buf, sem) in enumerate(((buf0, sem0), (buf1, sem1))):
            c_prev = 2 * p - 2 + b
            c_new = 2 * p + b
            dma(buf, sem, c_prev).wait()
            scatter(buf, c_prev, zf)
            scatter(buf, c_new, ones)
            dma(buf, sem, c_new).start()
        return carry

    lax.fori_loop(1, nch // 2, pair, 0)
    dma(buf0, sem0, nch - 2).wait()
    dma(buf1, sem1, nch - 1).wait()


def kernel(x):
    b, f = x.shape
    n = b * f
    call = pl.kernel(
        _sc_body,
        out_type=jax.ShapeDtypeStruct((n * VOCAB,), jnp.float32),
        mesh=plsc.VectorSubcoreMesh(
            core_axis_name="c", subcore_axis_name="s",
            num_cores=NC, num_subcores=NS),
        scratch_types=[
            pltpu.VMEM((n // NW,), jnp.int32),
            pltpu.VMEM((BUFW,), jnp.float32),
            pltpu.VMEM((BUFW,), jnp.float32),
            pltpu.SemaphoreType.DMA,
            pltpu.SemaphoreType.DMA,
        ],
        compiler_params=pltpu.CompilerParams(needs_layout_passes=False),
    )
    out = call(x.reshape(n), jnp.zeros((BUFW,), jnp.float32))
    return out.reshape(b, f, VOCAB)


# speedup vs baseline: 1.9577x; 1.9577x over previous
import jax
import jax.numpy as jnp
from jax import lax
from jax.experimental import pallas as pl
from jax.experimental.pallas import tpu as pltpu
from jax.experimental.pallas import tpu_sc as plsc

VOCAB = 1000
NC, NS = 2, 16
NW = NC * NS


def _sc_body(x_hbm, z_hbm, out_hbm, idx_v, buf0, buf1, sem0, sem1):
    n_batch = out_hbm.shape[0]
    nf = out_hbm.shape[1]
    bpw = n_batch // NW
    wid = lax.axis_index("s") * NC + lax.axis_index("c")
    base_b = wid * bpw

    pltpu.sync_copy(x_hbm.at[pl.ds(base_b * nf, bpw * nf)],
                    idx_v.at[pl.ds(0, bpw * nf)])
    pltpu.async_copy(z_hbm, buf0, sem0).wait()
    pltpu.async_copy(z_hbm, buf1, sem1).wait()

    lane = lax.iota(jnp.int32, 16)
    zero16 = jnp.zeros((16,), jnp.int32)
    ones = jnp.full((16,), 1.0, jnp.float32)
    zf = jnp.zeros((16,), jnp.float32)
    mask_hi = lane < (nf - 16)

    def scatter(buf, c, val):
        col_lo = plsc.load_gather(idx_v, [lane + c * nf])
        col_hi = plsc.load_gather(idx_v, [lane + (c * nf + 16)])
        plsc.store_scatter(buf, [zero16, lane, col_lo], val)
        plsc.store_scatter(buf, [zero16, lane + 16, col_hi], val,
                           mask=mask_hi)

    def dma(buf, sem, c):
        return pltpu.make_async_copy(
            buf, out_hbm.at[pl.ds(base_b + c, 1)], sem)

    scatter(buf0, 0, ones)
    dma(buf0, sem0, 0).start()
    scatter(buf1, 1, ones)
    dma(buf1, sem1, 1).start()

    def pair(p, carry):
        for b, (buf, sem) in enumerate(((buf0, sem0), (buf1, sem1))):
            c_prev = 2 * p - 2 + b
            c_new = 2 * p + b
            dma(buf, sem, c_prev).wait()
            scatter(buf, c_prev, zf)
            scatter(buf, c_new, ones)
            dma(buf, sem, c_new).start()
        return carry

    lax.fori_loop(1, bpw // 2, pair, 0)
    dma(buf0, sem0, bpw - 2).wait()
    dma(buf1, sem1, bpw - 1).wait()


def kernel(x):
    b, f = x.shape
    n = b * f
    call = pl.kernel(
        _sc_body,
        out_type=jax.ShapeDtypeStruct((b, f, VOCAB), jnp.float32),
        mesh=plsc.VectorSubcoreMesh(
            core_axis_name="c", subcore_axis_name="s",
            num_cores=NC, num_subcores=NS),
        scratch_types=[
            pltpu.VMEM((n // NW + 16,), jnp.int32),
            pltpu.VMEM((1, f, VOCAB), jnp.float32),
            pltpu.VMEM((1, f, VOCAB), jnp.float32),
            pltpu.SemaphoreType.DMA,
            pltpu.SemaphoreType.DMA,
        ],
        compiler_params=pltpu.CompilerParams(
            needs_layout_passes=False, use_tc_tiling_on_sc=True),
    )
    out = call(x.reshape(n), jnp.zeros((1, f, VOCAB), jnp.float32))
    return out


if __name__ == "__main__":
    pass


# speedup vs baseline: 2.0620x; 1.0532x over previous
import jax
import jax.numpy as jnp
from jax import lax
from jax.experimental import pallas as pl
from jax.experimental.pallas import tpu as pltpu

VOCAB = 1000
B_BLK = 32
NBUF = 8


def _onehot_block(x_ref, o_hbm, scratch, sems):
    i = pl.program_id(0)
    g = pl.num_programs(0)
    f = x_ref.shape[1]

    idx = x_ref[...]
    iota = lax.broadcasted_iota(jnp.int32, (B_BLK, f, VOCAB), 2)
    block = jnp.where(iota == idx[:, :, None], 1.0, 0.0).astype(jnp.float32)

    for j in range(NBUF):
        @pl.when(lax.rem(i, NBUF) == j)
        def _():
            @pl.when(i >= NBUF)
            def _():
                pltpu.make_async_copy(
                    scratch.at[j],
                    o_hbm.at[pl.ds((i - NBUF) * B_BLK, B_BLK)],
                    sems.at[j],
                ).wait()

            scratch[j] = block
            pltpu.make_async_copy(
                scratch.at[j],
                o_hbm.at[pl.ds(i * B_BLK, B_BLK)],
                sems.at[j],
            ).start(priority=j % 2)

    @pl.when(i == g - 1)
    def _():
        for j in range(NBUF):
            pltpu.make_async_copy(
                scratch.at[j],
                o_hbm.at[pl.ds((g - NBUF + j) * B_BLK, B_BLK)],
                sems.at[j],
            ).wait()


def kernel(x):
    b, f = x.shape
    grid = b // B_BLK
    return pl.pallas_call(
        _onehot_block,
        grid=(grid,),
        in_specs=[pl.BlockSpec((B_BLK, f), lambda i: (i, 0))],
        out_specs=pl.BlockSpec(memory_space=pl.ANY),
        out_shape=jax.ShapeDtypeStruct((b, f, VOCAB), jnp.float32),
        scratch_shapes=[
            pltpu.VMEM((NBUF, B_BLK, f, VOCAB), jnp.float32),
            pltpu.SemaphoreType.DMA((NBUF,)),
        ],
    )(x)
